# baseline (speedup 1.0000x reference)
import functools
import math

import jax
import jax.numpy as jnp
from jax import lax
from jax.experimental import pallas as pl
from jax.experimental.pallas import tpu as pltpu
from jax.experimental.pallas import tpu_sc as plsc

_NC = 2
_NS = 16
_NW = _NC * _NS

_CHUNK = 128
_GB = 8


def _gather_body(table_hbm, idx_hbm, out_hbm, idx_v, rows_v, tbuf, sem):
    wid = lax.axis_index("s") * _NC + lax.axis_index("c")
    n_chunks = idx_hbm.shape[0]
    n_ct = out_hbm.shape[2]
    per_w = n_chunks // _NW
    lanes = lax.iota(jnp.int32, 16)

    def do_chunk(c, j):
        pltpu.async_copy(table_hbm.at[idx_v.at[j]], rows_v, sem).wait()

        def trans_e(e, carry):
            for g in range(8):
                rows = g * 16 + lanes
                cols = jnp.full((16,), e, dtype=jnp.int32)
                vals = plsc.load_gather(rows_v, [rows, cols])
                tbuf[e, pl.ds(g * 16, 16)] = vals
            return carry

        lax.fori_loop(0, 32, trans_e, 0)

        l = c // n_ct
        ct = c % n_ct
        copies = [
            pltpu.async_copy(
                tbuf.at[pl.ds(sr * 8, 8)], out_hbm.at[l, sr, ct], sem
            )
            for sr in range(4)
        ]
        for cp in copies:
            cp.wait()
        return ()

    def step(i, carry):
        base = wid * per_w + i * _GB
        pltpu.sync_copy(idx_hbm.at[pl.ds(base, _GB)], idx_v)
        for j in range(_GB):
            do_chunk(base + j, j)
        return carry

    lax.fori_loop(0, per_w // _GB, step, 0)


def _make_gather(n_chunks, hist, n_ct):
    mesh = plsc.VectorSubcoreMesh(
        core_axis_name="c", subcore_axis_name="s",
        num_cores=_NC, num_subcores=_NS,
    )
    return pl.kernel(
        _gather_body,
        out_type=jax.ShapeDtypeStruct((hist, 4, n_ct, 8, 128), jnp.float32),
        mesh=mesh,
        scratch_types=[
            pltpu.VMEM((_GB, _CHUNK), jnp.int32),
            pltpu.VMEM((_CHUNK, 32), jnp.float32),
            pltpu.VMEM((32, _CHUNK), jnp.float32),
            pltpu.SemaphoreType.DMA,
        ],
        compiler_params=pltpu.CompilerParams(
            use_tc_tiling_on_sc=False, needs_layout_passes=False
        ),
    )


def _proj_body(e_ref, wt_ref, b_ref, out_ref):
    out_ref[0] = jnp.dot(
        wt_ref[...], e_ref[0], preferred_element_type=jnp.float32
    ) + b_ref[...]


def _proj(emb3, wt, b8):
    hist, e, bb = emb3.shape
    d = wt.shape[0]
    return pl.pallas_call(
        _proj_body,
        grid=(hist,),
        in_specs=[
            pl.BlockSpec((1, e, bb), lambda i: (i, 0, 0)),
            pl.BlockSpec((d, e), lambda i: (0, 0)),
            pl.BlockSpec((d, 1), lambda i: (0, 0)),
        ],
        out_specs=pl.BlockSpec((1, d, bb), lambda i: (i, 0, 0)),
        out_shape=jax.ShapeDtypeStruct((hist, d, bb), jnp.float32),
        compiler_params=pltpu.CompilerParams(
            dimension_semantics=("arbitrary",),
        ),
    )(emb3, wt, b8)


def kernel(x, table, W, b):
    batch, hist = x.shape
    n = batch * hist
    embed = table.shape[1]
    d_model = W.shape[1]
    scale = math.sqrt(float(d_model))

    n_ct = batch // _CHUNK
    idx = x.T.reshape(n // _CHUNK, _CHUNK).astype(jnp.int32)
    emb4 = _make_gather(n // _CHUNK, hist, n_ct)(table, idx)

    emb3 = (
        emb4.transpose(0, 1, 3, 2, 4).reshape(hist, embed, batch)
    )

    wt = (W * scale).T
    b8 = (b * scale).reshape(d_model, 1)
    out3 = _proj(emb3, wt, b8)
    return jnp.transpose(out3, (2, 0, 1))


# speedup vs baseline: 1.0955x; 1.0955x over previous
import functools
import math

import jax
import jax.numpy as jnp
from jax import lax
from jax.experimental import pallas as pl
from jax.experimental.pallas import tpu as pltpu
from jax.experimental.pallas import tpu_sc as plsc

_NC = 2
_NS = 16
_NW = _NC * _NS

_CHUNK = 128


def _transpose_chunk(rows_v, tbuf):
    lanes = lax.iota(jnp.int32, 16)
    for e in range(32):
        for g in range(8):
            rows = g * 16 + lanes
            cols = jnp.full((16,), e, dtype=jnp.int32)
            vals = plsc.load_gather(rows_v, [rows, cols])
            tbuf[e // 8, e % 8, pl.ds(g * 16, 16)] = vals


def _store_chunk(tbuf, out_hbm, l, ct, sem):
    for sr in range(4):
        pltpu.async_copy(tbuf.at[sr], out_hbm.at[l, sr, ct], sem)


def _drain_store(tbuf, out_hbm, sem):
    pltpu.make_async_copy(tbuf, out_hbm.at[0, :, 0], sem).wait()


def _gather_body(table_hbm, idx_hbm, out_hbm, idx_all, rows_a, rows_b,
                 tbuf_a, tbuf_b, sem_ga, sem_gb, sem_sa, sem_sb):
    wid = lax.axis_index("s") * _NC + lax.axis_index("c")
    n_chunks = idx_hbm.shape[0]
    n_ct = out_hbm.shape[2]
    per_w = n_chunks // _NW
    steps = per_w // 2
    base = wid * per_w

    pltpu.sync_copy(idx_hbm.at[pl.ds(base, per_w)], idx_all)
    pltpu.async_copy(table_hbm.at[idx_all.at[0]], rows_a, sem_ga)

    def step(i, carry):
        c0 = i * 2
        g0 = base + c0
        l0 = g0 // n_ct
        t0 = g0 % n_ct
        g1 = g0 + 1
        l1 = g1 // n_ct
        t1 = g1 % n_ct

        pltpu.async_copy(table_hbm.at[idx_all.at[c0 + 1]], rows_b, sem_gb)

        pltpu.make_async_copy(table_hbm.at[idx_all.at[c0]], rows_a,
                              sem_ga).wait()

        @pl.when(i > 0)
        def _():
            _drain_store(tbuf_a, out_hbm, sem_sa)

        _transpose_chunk(rows_a, tbuf_a)
        _store_chunk(tbuf_a, out_hbm, l0, t0, sem_sa)

        @pl.when(i < steps - 1)
        def _():
            pltpu.async_copy(table_hbm.at[idx_all.at[c0 + 2]], rows_a,
                             sem_ga)

        pltpu.make_async_copy(table_hbm.at[idx_all.at[c0 + 1]], rows_b,
                              sem_gb).wait()

        @pl.when(i > 0)
        def _():
            _drain_store(tbuf_b, out_hbm, sem_sb)

        _transpose_chunk(rows_b, tbuf_b)
        _store_chunk(tbuf_b, out_hbm, l1, t1, sem_sb)
        return carry

    lax.fori_loop(0, steps, step, 0)
    _drain_store(tbuf_a, out_hbm, sem_sa)
    _drain_store(tbuf_b, out_hbm, sem_sb)


def _make_gather(n_chunks, hist, n_ct):
    mesh = plsc.VectorSubcoreMesh(
        core_axis_name="c", subcore_axis_name="s",
        num_cores=_NC, num_subcores=_NS,
    )
    per_w = n_chunks // _NW
    return pl.kernel(
        _gather_body,
        out_type=jax.ShapeDtypeStruct((hist, 4, n_ct, 8, 128), jnp.float32),
        mesh=mesh,
        scratch_types=[
            pltpu.VMEM((per_w, _CHUNK), jnp.int32),
            pltpu.VMEM((_CHUNK, 32), jnp.float32),
            pltpu.VMEM((_CHUNK, 32), jnp.float32),
            pltpu.VMEM((4, 8, _CHUNK), jnp.float32),
            pltpu.VMEM((4, 8, _CHUNK), jnp.float32),
            pltpu.SemaphoreType.DMA,
            pltpu.SemaphoreType.DMA,
            pltpu.SemaphoreType.DMA,
            pltpu.SemaphoreType.DMA,
        ],
        compiler_params=pltpu.CompilerParams(
            use_tc_tiling_on_sc=False, needs_layout_passes=False
        ),
    )


def _proj_body(e_ref, wt_ref, b_ref, out_ref):
    out_ref[0] = jnp.dot(
        wt_ref[...], e_ref[0], preferred_element_type=jnp.float32
    ) + b_ref[...]


def _proj(emb3, wt, b8):
    hist, e, bb = emb3.shape
    d = wt.shape[0]
    return pl.pallas_call(
        _proj_body,
        grid=(hist,),
        in_specs=[
            pl.BlockSpec((1, e, bb), lambda i: (i, 0, 0)),
            pl.BlockSpec((d, e), lambda i: (0, 0)),
            pl.BlockSpec((d, 1), lambda i: (0, 0)),
        ],
        out_specs=pl.BlockSpec((1, d, bb), lambda i: (i, 0, 0)),
        out_shape=jax.ShapeDtypeStruct((hist, d, bb), jnp.float32),
        compiler_params=pltpu.CompilerParams(
            dimension_semantics=("arbitrary",),
        ),
    )(emb3, wt, b8)


def kernel(x, table, W, b):
    batch, hist = x.shape
    n = batch * hist
    embed = table.shape[1]
    d_model = W.shape[1]
    scale = math.sqrt(float(d_model))

    n_ct = batch // _CHUNK
    idx = x.T.reshape(n // _CHUNK, _CHUNK).astype(jnp.int32)
    emb4 = _make_gather(n // _CHUNK, hist, n_ct)(table, idx)

    emb3 = emb4.transpose(0, 1, 3, 2, 4).reshape(hist, embed, batch)

    wt = (W * scale).T
    b8 = (b * scale).reshape(d_model, 1)
    out3 = _proj(emb3, wt, b8)
    return jnp.transpose(out3, (2, 0, 1))


# speedup vs baseline: 1.8147x; 1.6565x over previous
import functools
import math

import jax
import jax.numpy as jnp
from jax import lax
from jax.experimental import pallas as pl
from jax.experimental.pallas import tpu as pltpu
from jax.experimental.pallas import tpu_sc as plsc

_NC = 2
_NS = 16
_NW = _NC * _NS

_CHUNK = 128
_GB = 8


def _gather_body(table_hbm, idx_hbm, out_hbm, idx_v, rows_v, sem):
    wid = lax.axis_index("s") * _NC + lax.axis_index("c")
    n_chunks = idx_hbm.shape[0]
    per_w = n_chunks // _NW

    def step(i, carry):
        base = wid * per_w + i * _GB
        pltpu.sync_copy(idx_hbm.at[pl.ds(base, _GB)], idx_v)
        copies = [
            pltpu.async_copy(table_hbm.at[idx_v.at[j]], rows_v.at[j], sem)
            for j in range(_GB)
        ]
        for c in copies:
            c.wait()
        pltpu.sync_copy(rows_v, out_hbm.at[pl.ds(base, _GB)])
        return carry

    lax.fori_loop(0, per_w // _GB, step, 0)


def _make_gather(n_chunks, embed):
    mesh = plsc.VectorSubcoreMesh(
        core_axis_name="c", subcore_axis_name="s",
        num_cores=_NC, num_subcores=_NS,
    )
    return pl.kernel(
        _gather_body,
        out_type=jax.ShapeDtypeStruct((n_chunks, _CHUNK, embed), jnp.float32),
        mesh=mesh,
        scratch_types=[
            pltpu.VMEM((_GB, _CHUNK), jnp.int32),
            pltpu.VMEM((_GB, _CHUNK, embed), jnp.float32),
            pltpu.SemaphoreType.DMA,
        ],
        compiler_params=pltpu.CompilerParams(use_tc_tiling_on_sc=False),
    )


def _proj_body(e_ref, wt_ref, b_ref, out_ref):
    e = e_ref[...]
    parts = []
    for k in range(4):
        ek = e[:, 32 * k:32 * (k + 1)]
        ok = lax.dot_general(
            wt_ref[...], ek,
            dimension_numbers=(((1,), (1,)), ((), ())),
            preferred_element_type=jnp.float32,
        )
        parts.append(ok)
    out_ref[0] = jnp.concatenate(parts, axis=1) + b_ref[...]


def _proj(emb_pk, wt, b8, hist, batch):
    rows = emb_pk.shape[0]
    rows_per_l = rows // hist
    d = wt.shape[0]
    return pl.pallas_call(
        _proj_body,
        grid=(hist,),
        in_specs=[
            pl.BlockSpec((rows_per_l, 128), lambda i: (i, 0)),
            pl.BlockSpec((d, 32), lambda i: (0, 0)),
            pl.BlockSpec((d, 1), lambda i: (0, 0)),
        ],
        out_specs=pl.BlockSpec((1, d, batch), lambda i: (i, 0, 0)),
        out_shape=jax.ShapeDtypeStruct((hist, d, batch), jnp.float32),
        compiler_params=pltpu.CompilerParams(
            dimension_semantics=("arbitrary",),
        ),
    )(emb_pk, wt, b8)


def kernel(x, table, W, b):
    batch, hist = x.shape
    n = batch * hist
    embed = table.shape[1]
    d_model = W.shape[1]
    scale = math.sqrt(float(d_model))
    quarter = batch // 4

    xp = x.T.reshape(hist, 4, quarter).transpose(0, 2, 1)
    idx = xp.reshape(n // _CHUNK, _CHUNK).astype(jnp.int32)
    emb = _make_gather(n // _CHUNK, embed)(table, idx)

    emb_pk = emb.reshape(n // 4, 128)

    wt = (W * scale).T
    b8 = (b * scale).reshape(d_model, 1)
    out3 = _proj(emb_pk, wt, b8, hist, batch)
    return jnp.transpose(out3, (2, 0, 1))
